# baseline (speedup 1.0000x reference)
import functools

import jax
import jax.numpy as jnp
from jax import lax
from jax.experimental import pallas as pl
from jax.experimental.pallas import tpu as pltpu
from jax.experimental.pallas import tpu_sc as plsc

N = 10000
E = 320000
D = 128

NC, NS = 2, 16
NW = NC * NS
EPW = E // NW
CH = 128
EPWP = 10240
NCH = EPWP // CH
PAD = EPWP - EPW
NP = 10112
RPT = NP // NS
NPC = 10240
RPTC = NPC // NS

_MESH = plsc.VectorSubcoreMesh(core_axis_name="c", subcore_axis_name="s")


NBUF = 3


def _sc_agg_body(h_hbm, src_hbm, dst_hbm, z_hbm, out_hbm,
                 sbufs, dbufs, rbufs, acc, xsems, dsems, gsems, ssems):
    c = lax.axis_index("c")
    s = lax.axis_index("s")
    base = (s * NC + c) * EPWP
    r0 = s * RPT

    pltpu.sync_copy(z_hbm.at[pl.ds(r0, RPT)], acc.at[pl.ds(r0, RPT)])
    plsc.subcore_barrier()

    def rbuf(b):
        return rbufs.at[pl.ds(b * CH, CH)]

    def fire_idx(ci, b):
        off = base + ci * CH
        pltpu.async_copy(src_hbm.at[pl.ds(off, CH)], sbufs.at[b], xsems[b])
        pltpu.async_copy(dst_hbm.at[pl.ds(off, CH)], dbufs.at[b], dsems[b])

    def fire_gather(b):
        pltpu.make_async_copy(src_hbm.at[pl.ds(0, CH)], sbufs.at[b],
                              xsems[b]).wait()
        pltpu.async_copy(h_hbm.at[sbufs.at[b]], rbuf(b), gsems[b])

    def wait_in(b):
        pltpu.make_async_copy(dst_hbm.at[pl.ds(0, CH)], dbufs.at[b],
                              dsems[b]).wait()
        pltpu.make_async_copy(h_hbm.at[pl.ds(0, CH)], rbuf(b),
                              gsems[b]).wait()

    def wait_scat(b):
        pltpu.make_async_copy(rbuf(b), acc.at[pl.ds(0, CH)], ssems[b]).wait()

    def step(ci, b):
        wait_in(b)
        pltpu.async_copy(rbuf(b), acc.at[dbufs.at[b]], ssems[b], add=True)

        nb = (b + 2) % NBUF

        @pl.when(jnp.logical_and(ci >= 1, ci + 2 < NCH))
        def _():
            wait_scat(nb)
            fire_idx(ci + 2, nb)

        @pl.when(jnp.logical_and(ci < 1, ci + 2 < NCH))
        def _():
            fire_idx(ci + 2, nb)

        @pl.when(ci + 1 < NCH)
        def _():
            fire_gather((b + 1) % NBUF)

    fire_idx(0, 0)
    fire_idx(1, 1)
    fire_gather(0)

    def body(j, carry):
        for b in range(NBUF):
            step(NBUF * j + b, b)
        return carry

    lax.fori_loop(0, NCH // NBUF, body, 0)
    for k in range(NCH - NCH // NBUF * NBUF):
        ci = NCH // NBUF * NBUF + k
        step(jnp.int32(ci), ci % NBUF)

    for b in range(NBUF):
        wait_scat(b)

    plsc.subcore_barrier()
    pltpu.sync_copy(acc.at[pl.ds(r0, RPT)], out_hbm.at[c, pl.ds(r0, RPT)])


_AGG = pl.kernel(
    _sc_agg_body,
    mesh=_MESH,
    out_type=jax.ShapeDtypeStruct((NC, NP, D), jnp.float32),
    scratch_types=[
        pltpu.VMEM((NBUF, CH), jnp.int32),
        pltpu.VMEM((NBUF, CH), jnp.int32),
        pltpu.VMEM((NBUF * CH, D), jnp.float32),
        pltpu.VMEM_SHARED((NP, D), jnp.float32),
        [pltpu.SemaphoreType.DMA] * NBUF,
        [pltpu.SemaphoreType.DMA] * NBUF,
        [pltpu.SemaphoreType.DMA] * NBUF,
        [pltpu.SemaphoreType.DMA] * NBUF,
    ],
)


def _sc_counts_body(dst_hbm, z_hbm, out_hbm, dA, dB, ones_v, acc, dsA, dsB):
    c = lax.axis_index("c")
    s = lax.axis_index("s")
    base = (s * NC + c) * EPWP
    r0 = s * RPTC

    for k in range(CH // 16):
        ones_v[pl.ds(16 * k, 16)] = jnp.full((16,), 1.0, jnp.float32)
    pltpu.sync_copy(z_hbm.at[pl.ds(r0, RPTC)], acc.at[pl.ds(r0, RPTC)])
    plsc.subcore_barrier()

    def fire(ci, dbuf, dsem):
        pltpu.async_copy(dst_hbm.at[pl.ds(base + ci * CH, CH)], dbuf, dsem)

    def drain_scatter(dbuf, dsem):
        pltpu.make_async_copy(dst_hbm.at[pl.ds(0, CH)], dbuf, dsem).wait()
        pltpu.sync_copy(ones_v, acc.at[dbuf], add=True)

    fire(0, dA, dsA)

    def body(j, carry):
        c0 = 2 * j
        fire(c0 + 1, dB, dsB)
        drain_scatter(dA, dsA)

        @pl.when(j < NCH // 2 - 1)
        def _():
            fire(c0 + 2, dA, dsA)

        drain_scatter(dB, dsB)
        return carry

    lax.fori_loop(0, NCH // 2, body, 0)

    plsc.subcore_barrier()
    pltpu.sync_copy(acc.at[pl.ds(r0, RPTC)], out_hbm.at[c, pl.ds(r0, RPTC)])


_CNT = pl.kernel(
    _sc_counts_body,
    mesh=_MESH,
    out_type=jax.ShapeDtypeStruct((NC, NPC), jnp.float32),
    scratch_types=[
        pltpu.VMEM((CH,), jnp.int32),
        pltpu.VMEM((CH,), jnp.int32),
        pltpu.VMEM((CH,), jnp.float32),
        pltpu.VMEM_SHARED((NPC,), jnp.float32),
        pltpu.SemaphoreType.DMA,
        pltpu.SemaphoreType.DMA,
    ],
)


BN = 1000


def _tc1_body(p0_ref, p1_ref, c0_ref, c1_ref, x_ref, wl_ref, b_ref, wr_ref,
              h_ref, rc_ref):
    cnt = c0_ref[...] + c1_ref[...]
    rc = 1.0 / jnp.maximum(cnt, 1.0)
    mean = (p0_ref[0] + p1_ref[0]) * rc
    acc = jnp.dot(mean, wl_ref[...], preferred_element_type=jnp.float32)
    acc = acc + jnp.dot(x_ref[...], wr_ref[...], preferred_element_type=jnp.float32)
    acc = acc + b_ref[...]
    h_ref[...] = jnp.maximum(acc, 0.0)
    rc_ref[...] = rc


def _tc_layer1(p, c0, c1, x, Wl, b, Wr):
    return pl.pallas_call(
        _tc1_body,
        grid=(N // BN,),
        in_specs=[
            pl.BlockSpec((1, BN, D), lambda i: (0, i, 0)),
            pl.BlockSpec((1, BN, D), lambda i: (1, i, 0)),
            pl.BlockSpec((BN, 1), lambda i: (i, 0)),
            pl.BlockSpec((BN, 1), lambda i: (i, 0)),
            pl.BlockSpec((BN, D), lambda i: (i, 0)),
            pl.BlockSpec((D, D), lambda i: (0, 0)),
            pl.BlockSpec((1, D), lambda i: (0, 0)),
            pl.BlockSpec((D, D), lambda i: (0, 0)),
        ],
        out_specs=[
            pl.BlockSpec((BN, D), lambda i: (i, 0)),
            pl.BlockSpec((BN, 1), lambda i: (i, 0)),
        ],
        out_shape=[
            jax.ShapeDtypeStruct((N, D), jnp.float32),
            jax.ShapeDtypeStruct((N, 1), jnp.float32),
        ],
    )(p, p, c0, c1, x, Wl, b, Wr)


def _make_tc23_body(relu):
    def body(p0_ref, p1_ref, x_ref, rc_ref, wl_ref, b_ref, wr_ref, h_ref):
        mean = (p0_ref[0] + p1_ref[0]) * rc_ref[...]
        acc = jnp.dot(mean, wl_ref[...], preferred_element_type=jnp.float32)
        acc = acc + jnp.dot(x_ref[...], wr_ref[...], preferred_element_type=jnp.float32)
        acc = acc + b_ref[...]
        h_ref[...] = jnp.maximum(acc, 0.0) if relu else acc
    return body


def _tc_layer23(p, x, rc, Wl, b, Wr, relu):
    return pl.pallas_call(
        _make_tc23_body(relu),
        grid=(N // BN,),
        in_specs=[
            pl.BlockSpec((1, BN, D), lambda i: (0, i, 0)),
            pl.BlockSpec((1, BN, D), lambda i: (1, i, 0)),
            pl.BlockSpec((BN, D), lambda i: (i, 0)),
            pl.BlockSpec((BN, 1), lambda i: (i, 0)),
            pl.BlockSpec((D, D), lambda i: (0, 0)),
            pl.BlockSpec((1, D), lambda i: (0, 0)),
            pl.BlockSpec((D, D), lambda i: (0, 0)),
        ],
        out_specs=pl.BlockSpec((BN, D), lambda i: (i, 0)),
        out_shape=jax.ShapeDtypeStruct((N, D), jnp.float32),
    )(p, p, x, rc, Wl, b, Wr)


def _pad_edges(src, dst):
    srcw = src.reshape(NW, EPW)
    dstw = dst.reshape(NW, EPW)
    pad_ids = jnp.arange(NW * PAD, dtype=jnp.int32).reshape(NW, PAD)
    src_pad = pad_ids % N
    dst_pad = N + pad_ids % (NP - N)
    src_p = jnp.concatenate([srcw, src_pad], axis=1).reshape(-1)
    dst_p = jnp.concatenate([dstw, dst_pad], axis=1).reshape(-1)
    return src_p, dst_p


def kernel(x, edge_index, W1l, b1, W1r, W2l, b2, W2r, W3l, b3, W3r):
    src = edge_index[0].astype(jnp.int32)
    dst = edge_index[1].astype(jnp.int32)
    src_p, dst_p = _pad_edges(src, dst)

    z = jnp.zeros((NP, D), jnp.float32)
    z1 = jnp.zeros((NPC,), jnp.float32)

    cp = _CNT(dst_p, z1)
    p = _AGG(x, src_p, dst_p, z)
    h1, rc = _tc_layer1(p, cp[0, :N].reshape(N, 1), cp[1, :N].reshape(N, 1),
                        x, W1l, b1.reshape(1, D), W1r)

    p = _AGG(h1, src_p, dst_p, z)
    h2 = _tc_layer23(p, h1, rc, W2l, b2.reshape(1, D), W2r, relu=True)

    p = _AGG(h2, src_p, dst_p, z)
    h3 = _tc_layer23(p, h2, rc, W3l, b3.reshape(1, D), W3r, relu=False)
    return h3


# speedup vs baseline: 1.1574x; 1.1574x over previous
import functools

import jax
import jax.numpy as jnp
from jax import lax
from jax.experimental import pallas as pl
from jax.experimental.pallas import tpu as pltpu
from jax.experimental.pallas import tpu_sc as plsc

N = 10000
E = 320000
D = 128

NC, NS = 2, 16
NW = NC * NS
EPW = E // NW
CH = 128
EPWP = 10240
NCH = EPWP // CH
PAD = EPWP - EPW
NP = 10240
RPT = NP // NS

_MESH = plsc.VectorSubcoreMesh(core_axis_name="c", subcore_axis_name="s")


NBUF = 2


def _sc_agg_body(h_hbm, src_hbm, dst_hbm, z_hbm, out_hbm,
                 sidx, dA, dB, rowsA, rowsB, acc, gsA, gsB, dsA, dsB):
    c = lax.axis_index("c")
    s = lax.axis_index("s")
    base = (s * NC + c) * EPWP
    r0 = s * RPT

    pltpu.sync_copy(src_hbm.at[pl.ds(base, EPWP)], sidx)
    pltpu.sync_copy(z_hbm.at[pl.ds(r0, RPT)], acc.at[pl.ds(r0, RPT)])
    plsc.subcore_barrier()

    def fire(ci, dbuf, rbuf, dsem, gsem):
        pltpu.async_copy(dst_hbm.at[pl.ds(base + ci * CH, CH)], dbuf, dsem)
        pltpu.async_copy(h_hbm.at[sidx.at[pl.ds(ci * CH, CH)]], rbuf, gsem)

    def drain_scatter(dbuf, rbuf, dsem, gsem):
        pltpu.make_async_copy(dst_hbm.at[pl.ds(0, CH)], dbuf, dsem).wait()
        pltpu.make_async_copy(h_hbm.at[pl.ds(0, CH)], rbuf, gsem).wait()
        pltpu.sync_copy(rbuf, acc.at[dbuf], add=True)

    fire(0, dA, rowsA, dsA, gsA)

    def body(j, carry):
        c0 = 2 * j
        fire(c0 + 1, dB, rowsB, dsB, gsB)
        drain_scatter(dA, rowsA, dsA, gsA)

        @pl.when(j < NCH // 2 - 1)
        def _():
            fire(c0 + 2, dA, rowsA, dsA, gsA)

        drain_scatter(dB, rowsB, dsB, gsB)
        return carry

    lax.fori_loop(0, NCH // 2, body, 0)

    plsc.subcore_barrier()
    pltpu.sync_copy(acc.at[pl.ds(r0, RPT)], out_hbm.at[c, pl.ds(r0, RPT)])


_AGG = pl.kernel(
    _sc_agg_body,
    mesh=_MESH,
    out_type=jax.ShapeDtypeStruct((NC, NP, D), jnp.float32),
    scratch_types=[
        pltpu.VMEM((EPWP,), jnp.int32),
        pltpu.VMEM((CH,), jnp.int32),
        pltpu.VMEM((CH,), jnp.int32),
        pltpu.VMEM((CH, D), jnp.float32),
        pltpu.VMEM((CH, D), jnp.float32),
        pltpu.VMEM_SHARED((NP, D), jnp.float32),
        pltpu.SemaphoreType.DMA,
        pltpu.SemaphoreType.DMA,
        pltpu.SemaphoreType.DMA,
        pltpu.SemaphoreType.DMA,
    ],
)


def _sc_counts_body(dst_hbm, z_hbm, out_hbm, dA, dB, ones_v, acc, dsA, dsB):
    c = lax.axis_index("c")
    s = lax.axis_index("s")
    base = (s * NC + c) * EPWP
    r0 = s * RPT

    for k in range(CH // 16):
        ones_v[pl.ds(16 * k, 16)] = jnp.full((16,), 1.0, jnp.float32)
    pltpu.sync_copy(z_hbm.at[pl.ds(r0, RPT)], acc.at[pl.ds(r0, RPT)])
    plsc.subcore_barrier()

    def fire(ci, dbuf, dsem):
        pltpu.async_copy(dst_hbm.at[pl.ds(base + ci * CH, CH)], dbuf, dsem)

    def drain_scatter(dbuf, dsem):
        pltpu.make_async_copy(dst_hbm.at[pl.ds(0, CH)], dbuf, dsem).wait()
        pltpu.sync_copy(ones_v, acc.at[dbuf], add=True)

    fire(0, dA, dsA)

    def body(j, carry):
        c0 = 2 * j
        fire(c0 + 1, dB, dsB)
        drain_scatter(dA, dsA)

        @pl.when(j < NCH // 2 - 1)
        def _():
            fire(c0 + 2, dA, dsA)

        drain_scatter(dB, dsB)
        return carry

    lax.fori_loop(0, NCH // 2, body, 0)

    plsc.subcore_barrier()
    pltpu.sync_copy(acc.at[pl.ds(r0, RPT)], out_hbm.at[c, pl.ds(r0, RPT)])


_CNT = pl.kernel(
    _sc_counts_body,
    mesh=_MESH,
    out_type=jax.ShapeDtypeStruct((NC, NP), jnp.float32),
    scratch_types=[
        pltpu.VMEM((CH,), jnp.int32),
        pltpu.VMEM((CH,), jnp.int32),
        pltpu.VMEM((CH,), jnp.float32),
        pltpu.VMEM_SHARED((NP,), jnp.float32),
        pltpu.SemaphoreType.DMA,
        pltpu.SemaphoreType.DMA,
    ],
)


BN = 1000


def _tc1_body(p0_ref, p1_ref, c0_ref, c1_ref, x_ref, wl_ref, b_ref, wr_ref,
              h_ref, rc_ref):
    cnt = c0_ref[...] + c1_ref[...]
    rc = 1.0 / jnp.maximum(cnt, 1.0)
    mean = (p0_ref[0] + p1_ref[0]) * rc
    acc = jnp.dot(mean, wl_ref[...], preferred_element_type=jnp.float32)
    acc = acc + jnp.dot(x_ref[...], wr_ref[...], preferred_element_type=jnp.float32)
    acc = acc + b_ref[...]
    h_ref[...] = jnp.maximum(acc, 0.0)
    rc_ref[...] = rc


def _tc_layer1(p, c0, c1, x, Wl, b, Wr):
    return pl.pallas_call(
        _tc1_body,
        grid=(N // BN,),
        in_specs=[
            pl.BlockSpec((1, BN, D), lambda i: (0, i, 0)),
            pl.BlockSpec((1, BN, D), lambda i: (1, i, 0)),
            pl.BlockSpec((BN, 1), lambda i: (i, 0)),
            pl.BlockSpec((BN, 1), lambda i: (i, 0)),
            pl.BlockSpec((BN, D), lambda i: (i, 0)),
            pl.BlockSpec((D, D), lambda i: (0, 0)),
            pl.BlockSpec((1, D), lambda i: (0, 0)),
            pl.BlockSpec((D, D), lambda i: (0, 0)),
        ],
        out_specs=[
            pl.BlockSpec((BN, D), lambda i: (i, 0)),
            pl.BlockSpec((BN, 1), lambda i: (i, 0)),
        ],
        out_shape=[
            jax.ShapeDtypeStruct((N, D), jnp.float32),
            jax.ShapeDtypeStruct((N, 1), jnp.float32),
        ],
    )(p, p, c0, c1, x, Wl, b, Wr)


def _make_tc23_body(relu):
    def body(p0_ref, p1_ref, x_ref, rc_ref, wl_ref, b_ref, wr_ref, h_ref):
        mean = (p0_ref[0] + p1_ref[0]) * rc_ref[...]
        acc = jnp.dot(mean, wl_ref[...], preferred_element_type=jnp.float32)
        acc = acc + jnp.dot(x_ref[...], wr_ref[...], preferred_element_type=jnp.float32)
        acc = acc + b_ref[...]
        h_ref[...] = jnp.maximum(acc, 0.0) if relu else acc
    return body


def _tc_layer23(p, x, rc, Wl, b, Wr, relu):
    return pl.pallas_call(
        _make_tc23_body(relu),
        grid=(N // BN,),
        in_specs=[
            pl.BlockSpec((1, BN, D), lambda i: (0, i, 0)),
            pl.BlockSpec((1, BN, D), lambda i: (1, i, 0)),
            pl.BlockSpec((BN, D), lambda i: (i, 0)),
            pl.BlockSpec((BN, 1), lambda i: (i, 0)),
            pl.BlockSpec((D, D), lambda i: (0, 0)),
            pl.BlockSpec((1, D), lambda i: (0, 0)),
            pl.BlockSpec((D, D), lambda i: (0, 0)),
        ],
        out_specs=pl.BlockSpec((BN, D), lambda i: (i, 0)),
        out_shape=jax.ShapeDtypeStruct((N, D), jnp.float32),
    )(p, p, x, rc, Wl, b, Wr)


def _pad_edges(src, dst):
    srcw = src.reshape(NW, EPW)
    dstw = dst.reshape(NW, EPW)
    pad_ids = jnp.arange(NW * PAD, dtype=jnp.int32).reshape(NW, PAD)
    src_pad = pad_ids % N
    dst_pad = N + pad_ids % (NP - N)
    src_p = jnp.concatenate([srcw, src_pad], axis=1).reshape(-1)
    dst_p = jnp.concatenate([dstw, dst_pad], axis=1).reshape(-1)
    return src_p, dst_p


def kernel(x, edge_index, W1l, b1, W1r, W2l, b2, W2r, W3l, b3, W3r):
    src = edge_index[0].astype(jnp.int32)
    dst = edge_index[1].astype(jnp.int32)
    src_p, dst_p = _pad_edges(src, dst)

    z = jnp.zeros((NP, D), jnp.float32)
    z1 = jnp.zeros((NP,), jnp.float32)

    cp = _CNT(dst_p, z1)
    p = _AGG(x, src_p, dst_p, z)
    h1, rc = _tc_layer1(p, cp[0, :N].reshape(N, 1), cp[1, :N].reshape(N, 1),
                        x, W1l, b1.reshape(1, D), W1r)

    p = _AGG(h1, src_p, dst_p, z)
    h2 = _tc_layer23(p, h1, rc, W2l, b2.reshape(1, D), W2r, relu=True)

    p = _AGG(h2, src_p, dst_p, z)
    h3 = _tc_layer23(p, h2, rc, W3l, b3.reshape(1, D), W3r, relu=False)
    return h3


# speedup vs baseline: 1.1969x; 1.0341x over previous
import functools

import jax
import jax.numpy as jnp
from jax import lax
from jax.experimental import pallas as pl
from jax.experimental.pallas import tpu as pltpu
from jax.experimental.pallas import tpu_sc as plsc

N = 10000
E = 320000
D = 128

NC, NS = 2, 16
NW = NC * NS
EPW = E // NW
CH = 128
EPWP = 10240
NCH = EPWP // CH
PAD = EPWP - EPW
NP = 10240
RPT = NP // NS

_MESH = plsc.VectorSubcoreMesh(core_axis_name="c", subcore_axis_name="s")


NBUF = 2


def _make_sc_agg(with_counts):
    out_type = [jax.ShapeDtypeStruct((NC, NP, D), jnp.float32)]
    scratch = [
        pltpu.VMEM((EPWP,), jnp.int32),
        pltpu.VMEM((CH,), jnp.int32),
        pltpu.VMEM((CH,), jnp.int32),
        pltpu.VMEM((CH, D), jnp.float32),
        pltpu.VMEM((CH, D), jnp.float32),
        pltpu.VMEM_SHARED((NP, D), jnp.float32),
        pltpu.SemaphoreType.DMA,
        pltpu.SemaphoreType.DMA,
        pltpu.SemaphoreType.DMA,
        pltpu.SemaphoreType.DMA,
    ]
    if with_counts:
        out_type.append(jax.ShapeDtypeStruct((NC, NP), jnp.float32))
        scratch += [
            pltpu.VMEM((CH,), jnp.float32),
            pltpu.VMEM_SHARED((NP,), jnp.float32),
        ]

    def agg(*args):
        it = iter(args)
        h_hbm, src_hbm, dst_hbm, z_hbm = (next(it) for _ in range(4))
        z1_hbm = next(it) if with_counts else None
        out_hbm = next(it)
        cnt_hbm = next(it) if with_counts else None
        sidx, dA, dB, rowsA, rowsB, acc = (next(it) for _ in range(6))
        gsA, gsB, dsA, dsB = (next(it) for _ in range(4))
        if with_counts:
            ones_v, acc1 = next(it), next(it)

        c = lax.axis_index("c")
        s = lax.axis_index("s")
        base = (s * NC + c) * EPWP
        r0 = s * RPT

        pltpu.sync_copy(src_hbm.at[pl.ds(base, EPWP)], sidx)
        pltpu.sync_copy(z_hbm.at[pl.ds(r0, RPT)], acc.at[pl.ds(r0, RPT)])
        if with_counts:
            for k in range(CH // 16):
                ones_v[pl.ds(16 * k, 16)] = jnp.full((16,), 1.0, jnp.float32)
            pltpu.sync_copy(z1_hbm.at[pl.ds(r0, RPT)], acc1.at[pl.ds(r0, RPT)])
        plsc.subcore_barrier()

        def fire(ci, dbuf, rbuf, dsem, gsem):
            pltpu.async_copy(dst_hbm.at[pl.ds(base + ci * CH, CH)], dbuf, dsem)
            pltpu.async_copy(h_hbm.at[sidx.at[pl.ds(ci * CH, CH)]], rbuf, gsem)

        def drain_scatter(dbuf, rbuf, dsem, gsem):
            pltpu.make_async_copy(dst_hbm.at[pl.ds(0, CH)], dbuf, dsem).wait()
            pltpu.make_async_copy(h_hbm.at[pl.ds(0, CH)], rbuf, gsem).wait()
            pltpu.sync_copy(rbuf, acc.at[dbuf], add=True)
            if with_counts:
                pltpu.sync_copy(ones_v, acc1.at[dbuf], add=True)

        fire(0, dA, rowsA, dsA, gsA)

        def body(j, carry):
            c0 = 2 * j
            fire(c0 + 1, dB, rowsB, dsB, gsB)
            drain_scatter(dA, rowsA, dsA, gsA)

            @pl.when(j < NCH // 2 - 1)
            def _():
                fire(c0 + 2, dA, rowsA, dsA, gsA)

            drain_scatter(dB, rowsB, dsB, gsB)
            return carry

        lax.fori_loop(0, NCH // 2, body, 0)

        plsc.subcore_barrier()
        pltpu.sync_copy(acc.at[pl.ds(r0, RPT)], out_hbm.at[c, pl.ds(r0, RPT)])
        if with_counts:
            pltpu.sync_copy(acc1.at[pl.ds(r0, RPT)],
                            cnt_hbm.at[c, pl.ds(r0, RPT)])

    return pl.kernel(agg, mesh=_MESH, out_type=out_type,
                     scratch_types=scratch)


_AGG1 = _make_sc_agg(with_counts=True)
_AGGN = _make_sc_agg(with_counts=False)


BN = 1000


def _tc1_body(p0_ref, p1_ref, c0_ref, c1_ref, x_ref, wl_ref, b_ref, wr_ref,
              h_ref, rc_ref):
    cnt = c0_ref[...] + c1_ref[...]
    rc = 1.0 / jnp.maximum(cnt, 1.0)
    mean = (p0_ref[0] + p1_ref[0]) * rc
    acc = jnp.dot(mean, wl_ref[...], preferred_element_type=jnp.float32)
    acc = acc + jnp.dot(x_ref[...], wr_ref[...], preferred_element_type=jnp.float32)
    acc = acc + b_ref[...]
    h_ref[...] = jnp.maximum(acc, 0.0)
    rc_ref[...] = rc


def _tc_layer1(p, c0, c1, x, Wl, b, Wr):
    return pl.pallas_call(
        _tc1_body,
        grid=(N // BN,),
        in_specs=[
            pl.BlockSpec((1, BN, D), lambda i: (0, i, 0)),
            pl.BlockSpec((1, BN, D), lambda i: (1, i, 0)),
            pl.BlockSpec((BN, 1), lambda i: (i, 0)),
            pl.BlockSpec((BN, 1), lambda i: (i, 0)),
            pl.BlockSpec((BN, D), lambda i: (i, 0)),
            pl.BlockSpec((D, D), lambda i: (0, 0)),
            pl.BlockSpec((1, D), lambda i: (0, 0)),
            pl.BlockSpec((D, D), lambda i: (0, 0)),
        ],
        out_specs=[
            pl.BlockSpec((BN, D), lambda i: (i, 0)),
            pl.BlockSpec((BN, 1), lambda i: (i, 0)),
        ],
        out_shape=[
            jax.ShapeDtypeStruct((N, D), jnp.float32),
            jax.ShapeDtypeStruct((N, 1), jnp.float32),
        ],
    )(p, p, c0, c1, x, Wl, b, Wr)


def _make_tc23_body(relu):
    def body(p0_ref, p1_ref, x_ref, rc_ref, wl_ref, b_ref, wr_ref, h_ref):
        mean = (p0_ref[0] + p1_ref[0]) * rc_ref[...]
        acc = jnp.dot(mean, wl_ref[...], preferred_element_type=jnp.float32)
        acc = acc + jnp.dot(x_ref[...], wr_ref[...], preferred_element_type=jnp.float32)
        acc = acc + b_ref[...]
        h_ref[...] = jnp.maximum(acc, 0.0) if relu else acc
    return body


def _tc_layer23(p, x, rc, Wl, b, Wr, relu):
    return pl.pallas_call(
        _make_tc23_body(relu),
        grid=(N // BN,),
        in_specs=[
            pl.BlockSpec((1, BN, D), lambda i: (0, i, 0)),
            pl.BlockSpec((1, BN, D), lambda i: (1, i, 0)),
            pl.BlockSpec((BN, D), lambda i: (i, 0)),
            pl.BlockSpec((BN, 1), lambda i: (i, 0)),
            pl.BlockSpec((D, D), lambda i: (0, 0)),
            pl.BlockSpec((1, D), lambda i: (0, 0)),
            pl.BlockSpec((D, D), lambda i: (0, 0)),
        ],
        out_specs=pl.BlockSpec((BN, D), lambda i: (i, 0)),
        out_shape=jax.ShapeDtypeStruct((N, D), jnp.float32),
    )(p, p, x, rc, Wl, b, Wr)


def _pad_edges(src, dst):
    srcw = src.reshape(NW, EPW)
    dstw = dst.reshape(NW, EPW)
    pad_ids = jnp.arange(NW * PAD, dtype=jnp.int32).reshape(NW, PAD)
    src_pad = pad_ids % N
    dst_pad = N + pad_ids % (NP - N)
    src_p = jnp.concatenate([srcw, src_pad], axis=1).reshape(-1)
    dst_p = jnp.concatenate([dstw, dst_pad], axis=1).reshape(-1)
    return src_p, dst_p


def kernel(x, edge_index, W1l, b1, W1r, W2l, b2, W2r, W3l, b3, W3r):
    src = edge_index[0].astype(jnp.int32)
    dst = edge_index[1].astype(jnp.int32)
    src_p, dst_p = _pad_edges(src, dst)

    z = jnp.zeros((NP, D), jnp.float32)
    z1 = jnp.zeros((NP,), jnp.float32)

    p, cp = _AGG1(x, src_p, dst_p, z, z1)
    h1, rc = _tc_layer1(p, cp[0, :N].reshape(N, 1), cp[1, :N].reshape(N, 1),
                        x, W1l, b1.reshape(1, D), W1r)

    p, = _AGGN(h1, src_p, dst_p, z)
    h2 = _tc_layer23(p, h1, rc, W2l, b2.reshape(1, D), W2r, relu=True)

    p, = _AGGN(h2, src_p, dst_p, z)
    h3 = _tc_layer23(p, h2, rc, W3l, b3.reshape(1, D), W3r, relu=False)
    return h3


# speedup vs baseline: 1.1983x; 1.0012x over previous
import jax
import jax.numpy as jnp
from jax import lax
from jax.experimental import pallas as pl
from jax.experimental.pallas import tpu as pltpu
from jax.experimental.pallas import tpu_sc as plsc

N = 10000
E = 320000
D = 128

NC, NS = 2, 16
NW = NC * NS
EPW = E // NW
CH = 128
EPWP = 10240
NCH = EPWP // CH
PAD = EPWP - EPW
NP = 10240
RPT = NP // NS

_MESH = plsc.VectorSubcoreMesh(core_axis_name="c", subcore_axis_name="s")


NBUF = 2


def _make_sc_agg(with_counts):
    out_type = [jax.ShapeDtypeStruct((NC, NP, D), jnp.float32)]
    scratch = [
        pltpu.VMEM((EPWP,), jnp.int32),
        pltpu.VMEM((CH,), jnp.int32),
        pltpu.VMEM((CH,), jnp.int32),
        pltpu.VMEM((CH, D), jnp.float32),
        pltpu.VMEM((CH, D), jnp.float32),
        pltpu.VMEM_SHARED((NP, D), jnp.float32),
        pltpu.SemaphoreType.DMA,
        pltpu.SemaphoreType.DMA,
        pltpu.SemaphoreType.DMA,
        pltpu.SemaphoreType.DMA,
    ]
    if with_counts:
        out_type.append(jax.ShapeDtypeStruct((NC, NP), jnp.float32))
        scratch += [
            pltpu.VMEM((CH,), jnp.float32),
            pltpu.VMEM_SHARED((NP,), jnp.float32),
        ]

    def agg(*args):
        it = iter(args)
        h_hbm, src_hbm, dst_hbm, z_hbm = (next(it) for _ in range(4))
        z1_hbm = next(it) if with_counts else None
        out_hbm = next(it)
        cnt_hbm = next(it) if with_counts else None
        sidx, dA, dB, rowsA, rowsB, acc = (next(it) for _ in range(6))
        gsA, gsB, dsA, dsB = (next(it) for _ in range(4))
        if with_counts:
            ones_v, acc1 = next(it), next(it)

        c = lax.axis_index("c")
        s = lax.axis_index("s")
        base = (s * NC + c) * EPWP
        r0 = s * RPT

        pltpu.sync_copy(src_hbm.at[pl.ds(base, EPWP)], sidx)
        pltpu.sync_copy(z_hbm.at[pl.ds(r0, RPT)], acc.at[pl.ds(r0, RPT)])
        if with_counts:
            for k in range(CH // 16):
                ones_v[pl.ds(16 * k, 16)] = jnp.full((16,), 1.0, jnp.float32)
            pltpu.sync_copy(z1_hbm.at[pl.ds(r0, RPT)], acc1.at[pl.ds(r0, RPT)])
        plsc.subcore_barrier()

        def fire(ci, dbuf, rbuf, dsem, gsem):
            pltpu.async_copy(dst_hbm.at[pl.ds(base + ci * CH, CH)], dbuf, dsem)
            pltpu.async_copy(h_hbm.at[sidx.at[pl.ds(ci * CH, CH)]], rbuf, gsem)

        def drain_scatter(dbuf, rbuf, dsem, gsem):
            pltpu.make_async_copy(dst_hbm.at[pl.ds(0, CH)], dbuf, dsem).wait()
            pltpu.make_async_copy(h_hbm.at[pl.ds(0, CH)], rbuf, gsem).wait()
            pltpu.sync_copy(rbuf, acc.at[dbuf], add=True)
            if with_counts:
                pltpu.sync_copy(ones_v, acc1.at[dbuf], add=True)

        fire(0, dA, rowsA, dsA, gsA)

        def body(j, carry):
            c0 = 2 * j
            fire(c0 + 1, dB, rowsB, dsB, gsB)
            drain_scatter(dA, rowsA, dsA, gsA)

            @pl.when(j < NCH // 2 - 1)
            def _():
                fire(c0 + 2, dA, rowsA, dsA, gsA)

            drain_scatter(dB, rowsB, dsB, gsB)
            return carry

        lax.fori_loop(0, NCH // 2, body, 0)

        plsc.subcore_barrier()
        pltpu.sync_copy(acc.at[pl.ds(r0, RPT)], out_hbm.at[c, pl.ds(r0, RPT)])
        if with_counts:
            pltpu.sync_copy(acc1.at[pl.ds(r0, RPT)],
                            cnt_hbm.at[c, pl.ds(r0, RPT)])

    return pl.kernel(agg, mesh=_MESH, out_type=out_type,
                     scratch_types=scratch)


_AGG1 = _make_sc_agg(with_counts=True)
_AGGN = _make_sc_agg(with_counts=False)


BN = 1000


def _tc1_body(p0_ref, p1_ref, c0_ref, c1_ref, x_ref, wl_ref, b_ref, wr_ref,
              h_ref, rc_ref):
    cnt = c0_ref[...] + c1_ref[...]
    rc = 1.0 / jnp.maximum(cnt, 1.0)
    mean = (p0_ref[0] + p1_ref[0]) * rc
    acc = jnp.dot(mean, wl_ref[...], preferred_element_type=jnp.float32)
    acc = acc + jnp.dot(x_ref[...], wr_ref[...], preferred_element_type=jnp.float32)
    acc = acc + b_ref[...]
    h_ref[...] = jnp.maximum(acc, 0.0)
    rc_ref[...] = rc


def _tc_layer1(p, c0, c1, x, Wl, b, Wr):
    return pl.pallas_call(
        _tc1_body,
        grid=(N // BN,),
        in_specs=[
            pl.BlockSpec((1, BN, D), lambda i: (0, i, 0)),
            pl.BlockSpec((1, BN, D), lambda i: (1, i, 0)),
            pl.BlockSpec((BN, 1), lambda i: (i, 0)),
            pl.BlockSpec((BN, 1), lambda i: (i, 0)),
            pl.BlockSpec((BN, D), lambda i: (i, 0)),
            pl.BlockSpec((D, D), lambda i: (0, 0)),
            pl.BlockSpec((1, D), lambda i: (0, 0)),
            pl.BlockSpec((D, D), lambda i: (0, 0)),
        ],
        out_specs=[
            pl.BlockSpec((BN, D), lambda i: (i, 0)),
            pl.BlockSpec((BN, 1), lambda i: (i, 0)),
        ],
        out_shape=[
            jax.ShapeDtypeStruct((N, D), jnp.float32),
            jax.ShapeDtypeStruct((N, 1), jnp.float32),
        ],
    )(p, p, c0, c1, x, Wl, b, Wr)


def _make_tc23_body(relu):
    def body(p0_ref, p1_ref, x_ref, rc_ref, wl_ref, b_ref, wr_ref, h_ref):
        mean = (p0_ref[0] + p1_ref[0]) * rc_ref[...]
        acc = jnp.dot(mean, wl_ref[...], preferred_element_type=jnp.float32)
        acc = acc + jnp.dot(x_ref[...], wr_ref[...], preferred_element_type=jnp.float32)
        acc = acc + b_ref[...]
        h_ref[...] = jnp.maximum(acc, 0.0) if relu else acc
    return body


def _tc_layer23(p, x, rc, Wl, b, Wr, relu):
    return pl.pallas_call(
        _make_tc23_body(relu),
        grid=(N // BN,),
        in_specs=[
            pl.BlockSpec((1, BN, D), lambda i: (0, i, 0)),
            pl.BlockSpec((1, BN, D), lambda i: (1, i, 0)),
            pl.BlockSpec((BN, D), lambda i: (i, 0)),
            pl.BlockSpec((BN, 1), lambda i: (i, 0)),
            pl.BlockSpec((D, D), lambda i: (0, 0)),
            pl.BlockSpec((1, D), lambda i: (0, 0)),
            pl.BlockSpec((D, D), lambda i: (0, 0)),
        ],
        out_specs=pl.BlockSpec((BN, D), lambda i: (i, 0)),
        out_shape=jax.ShapeDtypeStruct((N, D), jnp.float32),
    )(p, p, x, rc, Wl, b, Wr)


def _pad_edges(src, dst):
    srcw = src.reshape(NW, EPW)
    dstw = dst.reshape(NW, EPW)
    pad_ids = jnp.arange(NW * PAD, dtype=jnp.int32).reshape(NW, PAD)
    src_pad = pad_ids % N
    dst_pad = N + pad_ids % (NP - N)
    src_p = jnp.concatenate([srcw, src_pad], axis=1).reshape(-1)
    dst_p = jnp.concatenate([dstw, dst_pad], axis=1).reshape(-1)
    return src_p, dst_p


def kernel(x, edge_index, W1l, b1, W1r, W2l, b2, W2r, W3l, b3, W3r):
    src = edge_index[0].astype(jnp.int32)
    dst = edge_index[1].astype(jnp.int32)
    src_p, dst_p = _pad_edges(src, dst)

    z = jnp.zeros((NP, D), jnp.float32)
    z1 = jnp.zeros((NP,), jnp.float32)

    p, cp = _AGG1(x, src_p, dst_p, z, z1)
    h1, rc = _tc_layer1(p, cp[0, :N].reshape(N, 1), cp[1, :N].reshape(N, 1),
                        x, W1l, b1.reshape(1, D), W1r)

    p, = _AGGN(h1, src_p, dst_p, z)
    h2 = _tc_layer23(p, h1, rc, W2l, b2.reshape(1, D), W2r, relu=True)

    p, = _AGGN(h2, src_p, dst_p, z)
    h3 = _tc_layer23(p, h2, rc, W3l, b3.reshape(1, D), W3r, relu=False)
    return h3
